# Initial kernel scaffold; baseline (speedup 1.0000x reference)
import jax
import jax.numpy as jnp
from jax.experimental import pallas as pl


def kernel(x_audio, x_video, edge_index_aa, edge_index_vv, edge_index_va, batch_audio, batch_video, W_aa0, b_aa0, W_vv0, b_vv0, W_aa1, b_aa1, W_vv1, b_vv1, W_gs, W_gd, a_s, a_d, b_g, ln_a0_g, ln_a0_b, ln_v0_g, ln_v0_b, ln_a1_g, ln_a1_b, ln_v1_g, ln_v1_b, w_gate_a, w_gate_v, W_lin, b_lin):
    raise NotImplementedError("write your pallas kernel here")


# speedup vs baseline: 4.5475x; 4.5475x over previous
import functools

import jax
import jax.numpy as jnp
from jax import lax
from jax.experimental import pallas as pl
from jax.experimental.pallas import tpu as pltpu
from jax.experimental.pallas import tpu_sc as plsc

N = 10000
H = 128
G = 16
E = 320000
E_VA = 30000
NPAD = 10240
EVA_PAD = 30080
CHUNK = 128
NT = 16
RPT = NPAD // NT
AA_CHUNKS = E // CHUNK
AA_ITERS = -(-AA_CHUNKS // NT)
VA_CHUNKS = EVA_PAD // CHUNK
VA_ITERS = -(-VA_CHUNKS // NT)
BM = 1000


def _ln(x, g, b):
    mu = jnp.mean(x, axis=-1, keepdims=True)
    var = jnp.mean((x - mu) ** 2, axis=-1, keepdims=True)
    return (x - mu) / jnp.sqrt(var + 1e-5) * g + b


def _mm_body(x_ref, w_ref, o_ref):
    o_ref[...] = jnp.dot(x_ref[...], w_ref[...],
                         preferred_element_type=jnp.float32)


def _matmul(x, w):
    m, k = x.shape
    n = w.shape[1]
    return pl.pallas_call(
        _mm_body,
        grid=(m // BM,),
        in_specs=[pl.BlockSpec((BM, k), lambda i: (i, 0)),
                  pl.BlockSpec((k, n), lambda i: (0, 0))],
        out_specs=pl.BlockSpec((BM, n), lambda i: (i, 0)),
        out_shape=jax.ShapeDtypeStruct((m, n), jnp.float32),
    )(x, w)


def _sc_scatter_body(ya, yv, saa, daa, svv, dvv, zeros, out_a, out_v,
                     acc, idx_s, idx_d, rows, sem):
    cid = lax.axis_index("c")
    sid = lax.axis_index("s")

    pltpu.sync_copy(zeros.at[pl.ds(0, RPT)],
                    acc.at[pl.ds(sid * RPT, RPT)])
    plsc.subcore_barrier()

    def _edges(y_hbm, s_hbm, d_hbm):
        def body(i, _):
            chunk = i * NT + sid

            @pl.when(chunk < AA_CHUNKS)
            def _():
                base = chunk * CHUNK
                pltpu.sync_copy(s_hbm.at[pl.ds(base, CHUNK)], idx_s)
                pltpu.sync_copy(d_hbm.at[pl.ds(base, CHUNK)], idx_d)
                pltpu.async_copy(y_hbm.at[idx_s], rows, sem).wait()
                pltpu.sync_copy(rows, acc.at[idx_d], add=True)
            return 0
        lax.fori_loop(0, AA_ITERS, body, 0)

    @pl.when(cid == 0)
    def _():
        _edges(ya, saa, daa)

    @pl.when(cid == 1)
    def _():
        _edges(yv, svv, dvv)

    plsc.subcore_barrier()
    sl = pl.ds(sid * RPT, RPT)

    @pl.when(cid == 0)
    def _():
        pltpu.sync_copy(acc.at[sl], out_a.at[sl])

    @pl.when(cid == 1)
    def _():
        pltpu.sync_copy(acc.at[sl], out_v.at[sl])


_sc_scatter = pl.kernel(
    _sc_scatter_body,
    out_type=(jax.ShapeDtypeStruct((NPAD, H), jnp.float32),
              jax.ShapeDtypeStruct((NPAD, H), jnp.float32)),
    mesh=plsc.VectorSubcoreMesh(core_axis_name="c", subcore_axis_name="s"),
    scratch_types=[
        pltpu.VMEM_SHARED((NPAD, H), jnp.float32),
        pltpu.VMEM((CHUNK,), jnp.int32),
        pltpu.VMEM((CHUNK,), jnp.int32),
        pltpu.VMEM((CHUNK, H), jnp.float32),
        pltpu.SemaphoreType.DMA,
    ],
    compiler_params=pltpu.CompilerParams(needs_layout_passes=False),
    name="sc_edge_scatter",
)


def _sc_gat_body(hs_hbm, ss_hbm, sd_hbm, sva_hbm, dva_hbm, shift_hbm, zeros,
                 num_hbm, den_hbm,
                 acc, den_sh, ss_loc, sd_loc, shiftv, idx_s, idx_d, alpha,
                 rows, zv, sem):
    cid = lax.axis_index("c")
    sid = lax.axis_index("s")
    rpt = NPAD // NT

    pltpu.sync_copy(zeros.at[pl.ds(0, rpt)],
                    acc.at[pl.ds(sid * rpt, rpt)])

    def zbody(i, _):
        zv[pl.ds(i * 16, 16)] = jnp.zeros((16,), jnp.float32)
        return 0
    lax.fori_loop(0, rpt // 16, zbody, 0)
    pltpu.sync_copy(zv, den_sh.at[pl.ds(sid * rpt, rpt)])
    plsc.subcore_barrier()

    @pl.when(cid == 0)
    def _():
        pltpu.sync_copy(ss_hbm, ss_loc)
        pltpu.sync_copy(sd_hbm, sd_loc)
        pltpu.sync_copy(shift_hbm, shiftv)

        def body(i, _):
            chunk = i * NT + sid

            @pl.when(chunk < VA_CHUNKS)
            def _():
                base = chunk * CHUNK
                pltpu.sync_copy(sva_hbm.at[pl.ds(base, CHUNK)], idx_s)
                pltpu.sync_copy(dva_hbm.at[pl.ds(base, CHUNK)], idx_d)
                cp = pltpu.async_copy(hs_hbm.at[idx_s], rows, sem)
                sh = shiftv[...]
                for j in range(CHUNK // 16):
                    ii = idx_s[pl.ds(j * 16, 16)]
                    dd = idx_d[pl.ds(j * 16, 16)]
                    lg = plsc.load_gather(ss_loc, [ii]) \
                        + plsc.load_gather(sd_loc, [dd])
                    lg = jnp.maximum(lg, 0.2 * lg)
                    alpha[pl.ds(j * 16, 16)] = jnp.exp(lg - sh)
                pltpu.sync_copy(alpha, den_sh.at[idx_d], add=True)
                cp.wait()

                def rowgrp(j16, _):
                    a16 = alpha[pl.ds(j16 * 16, 16)]
                    lane = lax.iota(jnp.int32, 16)

                    def one(r2, _):
                        av = jnp.sum(jnp.where(lane == r2, a16, 0.0))
                        avv = jnp.full((16,), av, jnp.float32)
                        r = j16 * 16 + r2
                        for j2 in range(CHUNK // 16):
                            rows[r, pl.ds(j2 * 16, 16)] = (
                                rows[r, pl.ds(j2 * 16, 16)] * avv)
                        return 0
                    lax.fori_loop(0, 16, one, 0)
                    return 0
                lax.fori_loop(0, CHUNK // 16, rowgrp, 0)
                pltpu.sync_copy(rows, acc.at[idx_d], add=True)
            return 0
        lax.fori_loop(0, VA_ITERS, body, 0)

    plsc.subcore_barrier()

    @pl.when(cid == 0)
    def _():
        sl = pl.ds(sid * RPT, RPT)
        pltpu.sync_copy(acc.at[sl], num_hbm.at[sl])
        pltpu.sync_copy(den_sh.at[sl], den_hbm.at[sl])


_sc_gat = pl.kernel(
    _sc_gat_body,
    out_type=(jax.ShapeDtypeStruct((NPAD, H), jnp.float32),
              jax.ShapeDtypeStruct((NPAD,), jnp.float32)),
    mesh=plsc.VectorSubcoreMesh(core_axis_name="c", subcore_axis_name="s"),
    scratch_types=[
        pltpu.VMEM_SHARED((NPAD, H), jnp.float32),
        pltpu.VMEM_SHARED((NPAD,), jnp.float32),
        pltpu.VMEM((NPAD,), jnp.float32),
        pltpu.VMEM((NPAD,), jnp.float32),
        pltpu.VMEM((16,), jnp.float32),
        pltpu.VMEM((CHUNK,), jnp.int32),
        pltpu.VMEM((CHUNK,), jnp.int32),
        pltpu.VMEM((CHUNK,), jnp.float32),
        pltpu.VMEM((CHUNK, H), jnp.float32),
        pltpu.VMEM((NPAD // NT,), jnp.float32),
        pltpu.SemaphoreType.DMA,
    ],
    compiler_params=pltpu.CompilerParams(needs_layout_passes=False),
    name="sc_gat",
)


def _mid_body(ha_ref, hv_ref, baa0, bvv0, lnag, lnab, lnvg, lnvb,
              Waa1, Wvv1, Wgs, Wgd, asv, adv,
              ya1, yv1, hso, ssrc, sdst, mxs, mxd):
    i = pl.program_id(0)
    ha = jnp.maximum(_ln(ha_ref[...] + baa0[...], lnag[...], lnab[...]), 0.0)
    hv = jnp.maximum(_ln(hv_ref[...] + bvv0[...], lnvg[...], lnvb[...]), 0.0)
    ya1[...] = jnp.dot(ha, Waa1[...], preferred_element_type=jnp.float32)
    yv1[...] = jnp.dot(hv, Wvv1[...], preferred_element_type=jnp.float32)
    hs = jnp.dot(hv, Wgs[...], preferred_element_type=jnp.float32)
    hso[...] = hs
    hd = jnp.dot(ha, Wgd[...], preferred_element_type=jnp.float32)
    ss = jnp.dot(hs, asv[...], preferred_element_type=jnp.float32)
    sd = jnp.dot(hd, adv[...], preferred_element_type=jnp.float32)
    ssrc[...] = ss
    sdst[...] = sd

    @pl.when(i == 0)
    def _():
        mxs[...] = jnp.full_like(mxs[...], -1e30)
        mxd[...] = jnp.full_like(mxd[...], -1e30)
    mxs[...] = jnp.maximum(mxs[...], jnp.max(ss))
    mxd[...] = jnp.maximum(mxd[...], jnp.max(sd))


def _mid(ha_raw, hv_raw, baa0, bvv0, lnag, lnab, lnvg, lnvb,
         Waa1, Wvv1, Wgs, Wgd, asv, adv):
    full = lambda shape: pl.BlockSpec(shape, lambda i: (0, 0))
    blk = lambda shape: pl.BlockSpec(shape, lambda i: (i, 0))
    return pl.pallas_call(
        _mid_body,
        grid=(N // BM,),
        in_specs=[blk((BM, H)), blk((BM, H)),
                  full((1, H)), full((1, H)), full((1, H)), full((1, H)),
                  full((1, H)), full((1, H)),
                  full((H, H)), full((H, H)), full((H, H)), full((H, H)),
                  full((H, 1)), full((H, 1))],
        out_specs=[blk((BM, H)), blk((BM, H)), blk((BM, H)),
                   blk((BM, 1)), blk((BM, 1)),
                   full((8, 128)), full((8, 128))],
        out_shape=[jax.ShapeDtypeStruct((N, H), jnp.float32),
                   jax.ShapeDtypeStruct((N, H), jnp.float32),
                   jax.ShapeDtypeStruct((N, H), jnp.float32),
                   jax.ShapeDtypeStruct((N, 1), jnp.float32),
                   jax.ShapeDtypeStruct((N, 1), jnp.float32),
                   jax.ShapeDtypeStruct((8, 128), jnp.float32),
                   jax.ShapeDtypeStruct((8, 128), jnp.float32)],
    )(ha_raw, hv_raw, baa0, bvv0, lnag, lnab, lnvg, lnvb,
      Waa1, Wvv1, Wgs, Wgd, asv, adv)


def _readout(h, wg, batch):
    s = jnp.dot(h, wg, preferred_element_type=jnp.float32)
    gid = lax.broadcasted_iota(jnp.int32, (1, G), 1)
    mask = (batch == gid).astype(jnp.float32)
    M = jnp.max(jnp.where(mask > 0, s, -1e30), axis=0, keepdims=True)
    msel = jnp.sum(mask * M, axis=1, keepdims=True)
    e = jnp.exp(s - msel)
    S = jnp.sum(mask * e, axis=0, keepdims=True)
    ssel = jnp.sum(mask * S, axis=1, keepdims=True)
    gate = e / (ssel + 1e-16)
    w = mask * gate
    return lax.dot_general(w, h, (((0,), (0,)), ((), ())),
                           preferred_element_type=jnp.float32)


def _final_body(h1a, num, den, h1v, baa1, bg, bvv1, ln1ag, ln1ab,
                ln1vg, ln1vb, wga, wgv, ba, bv, Wl, bl, out):
    gat = num[...] / (den[...] + 1e-16)
    ha1 = jnp.maximum(
        _ln(h1a[...] + baa1[...] + gat + bg[...], ln1ag[...], ln1ab[...]),
        0.0)
    hv1 = jnp.maximum(_ln(h1v[...] + bvv1[...], ln1vg[...], ln1vb[...]), 0.0)
    ra = _readout(ha1, wga[...], ba[...])
    rv = _readout(hv1, wgv[...], bv[...])
    Wlv = Wl[...]
    out[...] = (jnp.dot(ra, Wlv[:H, :], preferred_element_type=jnp.float32)
                + jnp.dot(rv, Wlv[H:, :], preferred_element_type=jnp.float32)
                + bl[...])


def _final(h1a, num, den, h1v, baa1, bg, bvv1, ln1ag, ln1ab, ln1vg, ln1vb,
           wga, wgv, ba, bv, Wl, bl):
    return pl.pallas_call(
        _final_body,
        out_shape=jax.ShapeDtypeStruct((G, H), jnp.float32),
    )(h1a, num, den, h1v, baa1, bg, bvv1, ln1ag, ln1ab, ln1vg, ln1vb,
      wga, wgv, ba, bv, Wl, bl)


def kernel(x_audio, x_video, edge_index_aa, edge_index_vv, edge_index_va,
           batch_audio, batch_video, W_aa0, b_aa0, W_vv0, b_vv0, W_aa1,
           b_aa1, W_vv1, b_vv1, W_gs, W_gd, a_s, a_d, b_g, ln_a0_g, ln_a0_b,
           ln_v0_g, ln_v0_b, ln_a1_g, ln_a1_b, ln_v1_g, ln_v1_b, w_gate_a,
           w_gate_v, W_lin, b_lin):
    f32 = jnp.float32
    row = lambda v: v.reshape(1, H)
    col = lambda v: v.reshape(H, 1)
    saa, daa = edge_index_aa[0], edge_index_aa[1]
    svv, dvv = edge_index_vv[0], edge_index_vv[1]
    sva, dva = edge_index_va[0], edge_index_va[1]
    zeros = jnp.zeros((NPAD, H), f32)

    ya0 = _matmul(x_audio, W_aa0)
    yv0 = _matmul(x_video, W_vv0)
    ha_raw, hv_raw = _sc_scatter(ya0, yv0, saa, daa, svv, dvv, zeros)
    ha_raw, hv_raw = ha_raw[:N], hv_raw[:N]

    (ya1, yv1, hs, ssrc, sdst, mxs, mxd) = _mid(
        ha_raw, hv_raw, row(b_aa0), row(b_vv0), row(ln_a0_g), row(ln_a0_b),
        row(ln_v0_g), row(ln_v0_b), W_aa1, W_vv1, W_gs, W_gd,
        col(a_s), col(a_d))

    h1a_raw, h1v_raw = _sc_scatter(ya1, yv1, saa, daa, svv, dvv, zeros)
    h1a_raw, h1v_raw = h1a_raw[:N], h1v_raw[:N]

    ss_pad = jnp.concatenate([ssrc[:, 0], jnp.zeros((NPAD - N,), f32)])
    sd_pad = jnp.concatenate([sdst[:, 0], jnp.zeros((NPAD - N,), f32)])
    sva_p = jnp.concatenate(
        [sva, jnp.zeros((EVA_PAD - E_VA,), jnp.int32)])
    dva_p = jnp.concatenate(
        [dva, N + (jnp.arange(EVA_PAD - E_VA, dtype=jnp.int32)
                   % (NPAD - N))])
    bound = mxs[0, 0] + mxd[0, 0]
    shift = jnp.full((16,), jnp.maximum(bound, 0.2 * bound), f32)

    num, den = _sc_gat(hs, ss_pad, sd_pad, sva_p, dva_p, shift, zeros)
    num, den = num[:N], den[:N]

    out = _final(
        h1a_raw, num, den.reshape(N, 1), h1v_raw, row(b_aa1), row(b_g),
        row(b_vv1), row(ln_a1_g), row(ln_a1_b), row(ln_v1_g), row(ln_v1_b),
        col(w_gate_a), col(w_gate_v),
        batch_audio.reshape(N, 1), batch_video.reshape(N, 1), W_lin,
        row(b_lin))
    return out


# speedup vs baseline: 6.7882x; 1.4927x over previous
import functools

import jax
import jax.numpy as jnp
from jax import lax
from jax.experimental import pallas as pl
from jax.experimental.pallas import tpu as pltpu
from jax.experimental.pallas import tpu_sc as plsc

N = 10000
H = 128
G = 16
E = 320000
E_VA = 30000
NPAD = 10240
EVA_PAD = 30080
CHUNK = 128
NT = 16
RPT = NPAD // NT
AA_CHUNKS = E // CHUNK
AA_ITERS = -(-AA_CHUNKS // NT)
VA_CHUNKS = EVA_PAD // CHUNK
VA_ITERS = -(-VA_CHUNKS // NT)
BM = 1000


def _ln(x, g, b):
    mu = jnp.mean(x, axis=-1, keepdims=True)
    var = jnp.mean((x - mu) ** 2, axis=-1, keepdims=True)
    return (x - mu) / jnp.sqrt(var + 1e-5) * g + b


def _mm_body(x_ref, w_ref, o_ref):
    o_ref[...] = jnp.dot(x_ref[...], w_ref[...],
                         preferred_element_type=jnp.float32)


def _matmul(x, w):
    m, k = x.shape
    n = w.shape[1]
    return pl.pallas_call(
        _mm_body,
        grid=(m // BM,),
        in_specs=[pl.BlockSpec((BM, k), lambda i: (i, 0)),
                  pl.BlockSpec((k, n), lambda i: (0, 0))],
        out_specs=pl.BlockSpec((BM, n), lambda i: (i, 0)),
        out_shape=jax.ShapeDtypeStruct((m, n), jnp.float32),
    )(x, w)


def _sc_scatter_body(ya, yv, saa, daa, svv, dvv, zeros, out_a, out_v,
                     acc, idx_s0, idx_s1, idx_d0, idx_d1, rows0, rows1,
                     gsem0, gsem1):
    cid = lax.axis_index("c")
    sid = lax.axis_index("s")
    idx_s = (idx_s0, idx_s1)
    idx_d = (idx_d0, idx_d1)
    rows = (rows0, rows1)
    gsem = (gsem0, gsem1)

    pltpu.sync_copy(zeros.at[pl.ds(0, RPT)],
                    acc.at[pl.ds(sid * RPT, RPT)])
    plsc.subcore_barrier()

    def _edges(y_hbm, s_hbm, d_hbm):
        nj = (AA_CHUNKS - 1 - sid) // NT + 1

        def ld(j, b):
            base = (j * NT + sid) * CHUNK
            pltpu.sync_copy(s_hbm.at[pl.ds(base, CHUNK)], idx_s[b])
            pltpu.sync_copy(d_hbm.at[pl.ds(base, CHUNK)], idx_d[b])
            pltpu.async_copy(y_hbm.at[idx_s[b]], rows[b], gsem[b])

        ld(0, 0)

        def body(jo, _):
            for b in range(2):
                j = jo * 2 + b

                @pl.when(j < nj)
                def _():
                    @pl.when(j + 1 < nj)
                    def _():
                        ld(j + 1, 1 - b)
                    pltpu.make_async_copy(
                        y_hbm.at[idx_s[b]], rows[b], gsem[b]).wait()
                    pltpu.sync_copy(rows[b], acc.at[idx_d[b]], add=True)
            return 0
        lax.fori_loop(0, (AA_ITERS + 1) // 2, body, 0)

    @pl.when(cid == 0)
    def _():
        _edges(ya, saa, daa)

    @pl.when(cid == 1)
    def _():
        _edges(yv, svv, dvv)

    plsc.subcore_barrier()
    sl = pl.ds(sid * RPT, RPT)

    @pl.when(cid == 0)
    def _():
        pltpu.sync_copy(acc.at[sl], out_a.at[sl])

    @pl.when(cid == 1)
    def _():
        pltpu.sync_copy(acc.at[sl], out_v.at[sl])


_sc_scatter = pl.kernel(
    _sc_scatter_body,
    out_type=(jax.ShapeDtypeStruct((NPAD, H), jnp.float32),
              jax.ShapeDtypeStruct((NPAD, H), jnp.float32)),
    mesh=plsc.VectorSubcoreMesh(core_axis_name="c", subcore_axis_name="s"),
    scratch_types=[
        pltpu.VMEM_SHARED((NPAD, H), jnp.float32),
        pltpu.VMEM((CHUNK,), jnp.int32),
        pltpu.VMEM((CHUNK,), jnp.int32),
        pltpu.VMEM((CHUNK,), jnp.int32),
        pltpu.VMEM((CHUNK,), jnp.int32),
        pltpu.VMEM((CHUNK, H), jnp.float32),
        pltpu.VMEM((CHUNK, H), jnp.float32),
        pltpu.SemaphoreType.DMA,
        pltpu.SemaphoreType.DMA,
    ],
    compiler_params=pltpu.CompilerParams(needs_layout_passes=False),
    name="sc_edge_scatter",
)


def _sc_gat_body(hs_hbm, ss_hbm, sd_hbm, sva_hbm, dva_hbm, shift_hbm, zeros,
                 num_hbm, den_hbm,
                 acc, den_sh, ss_loc, sd_loc, shiftv, idx_s, idx_d, alpha,
                 rows, zv, sem):
    cid = lax.axis_index("c")
    sid = lax.axis_index("s")
    rpt = NPAD // NT

    pltpu.sync_copy(zeros.at[pl.ds(0, rpt)],
                    acc.at[pl.ds(sid * rpt, rpt)])

    def zbody(i, _):
        zv[pl.ds(i * 16, 16)] = jnp.zeros((16,), jnp.float32)
        return 0
    lax.fori_loop(0, rpt // 16, zbody, 0)
    pltpu.sync_copy(zv, den_sh.at[pl.ds(sid * rpt, rpt)])
    plsc.subcore_barrier()

    @pl.when(cid == 0)
    def _():
        pltpu.sync_copy(ss_hbm, ss_loc)
        pltpu.sync_copy(sd_hbm, sd_loc)
        pltpu.sync_copy(shift_hbm, shiftv)

        def body(i, _):
            chunk = i * NT + sid

            @pl.when(chunk < VA_CHUNKS)
            def _():
                base = chunk * CHUNK
                pltpu.sync_copy(sva_hbm.at[pl.ds(base, CHUNK)], idx_s)
                pltpu.sync_copy(dva_hbm.at[pl.ds(base, CHUNK)], idx_d)
                cp = pltpu.async_copy(hs_hbm.at[idx_s], rows, sem)
                sh = shiftv[...]
                for j in range(CHUNK // 16):
                    ii = idx_s[pl.ds(j * 16, 16)]
                    dd = idx_d[pl.ds(j * 16, 16)]
                    lg = plsc.load_gather(ss_loc, [ii]) \
                        + plsc.load_gather(sd_loc, [dd])
                    lg = jnp.maximum(lg, 0.2 * lg)
                    alpha[pl.ds(j * 16, 16)] = jnp.exp(lg - sh)
                pltpu.sync_copy(alpha, den_sh.at[idx_d], add=True)
                cp.wait()

                def rowgrp(j16, _):
                    a16 = alpha[pl.ds(j16 * 16, 16)]
                    lane = lax.iota(jnp.int32, 16)

                    def one(r2, _):
                        av = jnp.sum(jnp.where(lane == r2, a16, 0.0))
                        avv = jnp.full((16,), av, jnp.float32)
                        r = j16 * 16 + r2
                        for j2 in range(CHUNK // 16):
                            rows[r, pl.ds(j2 * 16, 16)] = (
                                rows[r, pl.ds(j2 * 16, 16)] * avv)
                        return 0
                    lax.fori_loop(0, 16, one, 0)
                    return 0
                lax.fori_loop(0, CHUNK // 16, rowgrp, 0)
                pltpu.sync_copy(rows, acc.at[idx_d], add=True)
            return 0
        lax.fori_loop(0, VA_ITERS, body, 0)

    plsc.subcore_barrier()

    @pl.when(cid == 0)
    def _():
        sl = pl.ds(sid * RPT, RPT)
        pltpu.sync_copy(acc.at[sl], num_hbm.at[sl])
        pltpu.sync_copy(den_sh.at[sl], den_hbm.at[sl])


_sc_gat = pl.kernel(
    _sc_gat_body,
    out_type=(jax.ShapeDtypeStruct((NPAD, H), jnp.float32),
              jax.ShapeDtypeStruct((NPAD,), jnp.float32)),
    mesh=plsc.VectorSubcoreMesh(core_axis_name="c", subcore_axis_name="s"),
    scratch_types=[
        pltpu.VMEM_SHARED((NPAD, H), jnp.float32),
        pltpu.VMEM_SHARED((NPAD,), jnp.float32),
        pltpu.VMEM((NPAD,), jnp.float32),
        pltpu.VMEM((NPAD,), jnp.float32),
        pltpu.VMEM((16,), jnp.float32),
        pltpu.VMEM((CHUNK,), jnp.int32),
        pltpu.VMEM((CHUNK,), jnp.int32),
        pltpu.VMEM((CHUNK,), jnp.float32),
        pltpu.VMEM((CHUNK, H), jnp.float32),
        pltpu.VMEM((NPAD // NT,), jnp.float32),
        pltpu.SemaphoreType.DMA,
    ],
    compiler_params=pltpu.CompilerParams(needs_layout_passes=False),
    name="sc_gat",
)


def _mid_body(ha_ref, hv_ref, baa0, bvv0, lnag, lnab, lnvg, lnvb,
              Waa1, Wvv1, Wgs, Wgd, asv, adv,
              ya1, yv1, hso, ssrc, sdst, mxs, mxd):
    i = pl.program_id(0)
    ha = jnp.maximum(_ln(ha_ref[...] + baa0[...], lnag[...], lnab[...]), 0.0)
    hv = jnp.maximum(_ln(hv_ref[...] + bvv0[...], lnvg[...], lnvb[...]), 0.0)
    ya1[...] = jnp.dot(ha, Waa1[...], preferred_element_type=jnp.float32)
    yv1[...] = jnp.dot(hv, Wvv1[...], preferred_element_type=jnp.float32)
    hs = jnp.dot(hv, Wgs[...], preferred_element_type=jnp.float32)
    hso[...] = hs
    hd = jnp.dot(ha, Wgd[...], preferred_element_type=jnp.float32)
    ss = jnp.dot(hs, asv[...], preferred_element_type=jnp.float32)
    sd = jnp.dot(hd, adv[...], preferred_element_type=jnp.float32)
    ssrc[...] = ss
    sdst[...] = sd

    @pl.when(i == 0)
    def _():
        mxs[...] = jnp.full_like(mxs[...], -1e30)
        mxd[...] = jnp.full_like(mxd[...], -1e30)
    mxs[...] = jnp.maximum(mxs[...], jnp.max(ss))
    mxd[...] = jnp.maximum(mxd[...], jnp.max(sd))


def _mid(ha_raw, hv_raw, baa0, bvv0, lnag, lnab, lnvg, lnvb,
         Waa1, Wvv1, Wgs, Wgd, asv, adv):
    full = lambda shape: pl.BlockSpec(shape, lambda i: (0, 0))
    blk = lambda shape: pl.BlockSpec(shape, lambda i: (i, 0))
    return pl.pallas_call(
        _mid_body,
        grid=(N // BM,),
        in_specs=[blk((BM, H)), blk((BM, H)),
                  full((1, H)), full((1, H)), full((1, H)), full((1, H)),
                  full((1, H)), full((1, H)),
                  full((H, H)), full((H, H)), full((H, H)), full((H, H)),
                  full((H, 1)), full((H, 1))],
        out_specs=[blk((BM, H)), blk((BM, H)), blk((BM, H)),
                   blk((BM, 1)), blk((BM, 1)),
                   full((8, 128)), full((8, 128))],
        out_shape=[jax.ShapeDtypeStruct((N, H), jnp.float32),
                   jax.ShapeDtypeStruct((N, H), jnp.float32),
                   jax.ShapeDtypeStruct((N, H), jnp.float32),
                   jax.ShapeDtypeStruct((N, 1), jnp.float32),
                   jax.ShapeDtypeStruct((N, 1), jnp.float32),
                   jax.ShapeDtypeStruct((8, 128), jnp.float32),
                   jax.ShapeDtypeStruct((8, 128), jnp.float32)],
    )(ha_raw, hv_raw, baa0, bvv0, lnag, lnab, lnvg, lnvb,
      Waa1, Wvv1, Wgs, Wgd, asv, adv)


def _readout(h, wg, batch):
    s = jnp.dot(h, wg, preferred_element_type=jnp.float32)
    gid = lax.broadcasted_iota(jnp.int32, (1, G), 1)
    mask = (batch == gid).astype(jnp.float32)
    M = jnp.max(jnp.where(mask > 0, s, -1e30), axis=0, keepdims=True)
    msel = jnp.sum(mask * M, axis=1, keepdims=True)
    e = jnp.exp(s - msel)
    S = jnp.sum(mask * e, axis=0, keepdims=True)
    ssel = jnp.sum(mask * S, axis=1, keepdims=True)
    gate = e / (ssel + 1e-16)
    w = mask * gate
    return lax.dot_general(w, h, (((0,), (0,)), ((), ())),
                           preferred_element_type=jnp.float32)


def _final_body(h1a, num, den, h1v, baa1, bg, bvv1, ln1ag, ln1ab,
                ln1vg, ln1vb, wga, wgv, ba, bv, Wl, bl, out):
    gat = num[...] / (den[...] + 1e-16)
    ha1 = jnp.maximum(
        _ln(h1a[...] + baa1[...] + gat + bg[...], ln1ag[...], ln1ab[...]),
        0.0)
    hv1 = jnp.maximum(_ln(h1v[...] + bvv1[...], ln1vg[...], ln1vb[...]), 0.0)
    ra = _readout(ha1, wga[...], ba[...])
    rv = _readout(hv1, wgv[...], bv[...])
    Wlv = Wl[...]
    out[...] = (jnp.dot(ra, Wlv[:H, :], preferred_element_type=jnp.float32)
                + jnp.dot(rv, Wlv[H:, :], preferred_element_type=jnp.float32)
                + bl[...])


def _final(h1a, num, den, h1v, baa1, bg, bvv1, ln1ag, ln1ab, ln1vg, ln1vb,
           wga, wgv, ba, bv, Wl, bl):
    return pl.pallas_call(
        _final_body,
        out_shape=jax.ShapeDtypeStruct((G, H), jnp.float32),
    )(h1a, num, den, h1v, baa1, bg, bvv1, ln1ag, ln1ab, ln1vg, ln1vb,
      wga, wgv, ba, bv, Wl, bl)


def kernel(x_audio, x_video, edge_index_aa, edge_index_vv, edge_index_va,
           batch_audio, batch_video, W_aa0, b_aa0, W_vv0, b_vv0, W_aa1,
           b_aa1, W_vv1, b_vv1, W_gs, W_gd, a_s, a_d, b_g, ln_a0_g, ln_a0_b,
           ln_v0_g, ln_v0_b, ln_a1_g, ln_a1_b, ln_v1_g, ln_v1_b, w_gate_a,
           w_gate_v, W_lin, b_lin):
    f32 = jnp.float32
    row = lambda v: v.reshape(1, H)
    col = lambda v: v.reshape(H, 1)
    saa, daa = edge_index_aa[0], edge_index_aa[1]
    svv, dvv = edge_index_vv[0], edge_index_vv[1]
    sva, dva = edge_index_va[0], edge_index_va[1]
    zeros = jnp.zeros((NPAD, H), f32)

    ya0 = _matmul(x_audio, W_aa0)
    yv0 = _matmul(x_video, W_vv0)
    ha_raw, hv_raw = _sc_scatter(ya0, yv0, saa, daa, svv, dvv, zeros)
    ha_raw, hv_raw = ha_raw[:N], hv_raw[:N]

    (ya1, yv1, hs, ssrc, sdst, mxs, mxd) = _mid(
        ha_raw, hv_raw, row(b_aa0), row(b_vv0), row(ln_a0_g), row(ln_a0_b),
        row(ln_v0_g), row(ln_v0_b), W_aa1, W_vv1, W_gs, W_gd,
        col(a_s), col(a_d))

    h1a_raw, h1v_raw = _sc_scatter(ya1, yv1, saa, daa, svv, dvv, zeros)
    h1a_raw, h1v_raw = h1a_raw[:N], h1v_raw[:N]

    ss_pad = jnp.concatenate([ssrc[:, 0], jnp.zeros((NPAD - N,), f32)])
    sd_pad = jnp.concatenate([sdst[:, 0], jnp.zeros((NPAD - N,), f32)])
    sva_p = jnp.concatenate(
        [sva, jnp.zeros((EVA_PAD - E_VA,), jnp.int32)])
    dva_p = jnp.concatenate(
        [dva, N + (jnp.arange(EVA_PAD - E_VA, dtype=jnp.int32)
                   % (NPAD - N))])
    bound = mxs[0, 0] + mxd[0, 0]
    shift = jnp.full((16,), jnp.maximum(bound, 0.2 * bound), f32)

    num, den = _sc_gat(hs, ss_pad, sd_pad, sva_p, dva_p, shift, zeros)
    num, den = num[:N], den[:N]

    out = _final(
        h1a_raw, num, den.reshape(N, 1), h1v_raw, row(b_aa1), row(b_g),
        row(b_vv1), row(ln_a1_g), row(ln_a1_b), row(ln_v1_g), row(ln_v1_b),
        col(w_gate_a), col(w_gate_v),
        batch_audio.reshape(N, 1), batch_video.reshape(N, 1), W_lin,
        row(b_lin))
    return out


# speedup vs baseline: 7.5706x; 1.1153x over previous
import functools

import jax
import jax.numpy as jnp
from jax import lax
from jax.experimental import pallas as pl
from jax.experimental.pallas import tpu as pltpu
from jax.experimental.pallas import tpu_sc as plsc

N = 10000
H = 128
G = 16
E = 320000
E_VA = 30000
NPAD = 10240
EVA_PAD = 30080
CHUNK = 128
NT = 16
RPT = NPAD // NT
E_P = 320512
CROWS = E_P // CHUNK
SUP = CROWS // 8
SUP_ITERS = (-(-SUP // NT) + 1) // 2
VA_CHUNKS = EVA_PAD // CHUNK
VA_ITERS = -(-VA_CHUNKS // NT)
BM = 1000


def _ln(x, g, b):
    mu = jnp.mean(x, axis=-1, keepdims=True)
    var = jnp.mean((x - mu) ** 2, axis=-1, keepdims=True)
    return (x - mu) / jnp.sqrt(var + 1e-5) * g + b


def _mm_body(x_ref, w_ref, o_ref):
    o_ref[...] = jnp.dot(x_ref[...], w_ref[...],
                         preferred_element_type=jnp.float32)


def _matmul(x, w):
    m, k = x.shape
    n = w.shape[1]
    return pl.pallas_call(
        _mm_body,
        grid=(m // BM,),
        in_specs=[pl.BlockSpec((BM, k), lambda i: (i, 0)),
                  pl.BlockSpec((k, n), lambda i: (0, 0))],
        out_specs=pl.BlockSpec((BM, n), lambda i: (i, 0)),
        out_shape=jax.ShapeDtypeStruct((m, n), jnp.float32),
    )(x, w)


def _sc_scatter_body(ya, yv, saa, daa, svv, dvv, zeros, out_a, out_v,
                     acc, idx_s0, idx_s1, idx_d0, idx_d1, rows0, rows1,
                     gsem0, gsem1, ssem0, ssem1):
    cid = lax.axis_index("c")
    sid = lax.axis_index("s")
    idx_s = (idx_s0, idx_s1)
    idx_d = (idx_d0, idx_d1)
    rows = (rows0, rows1)
    gsem = (gsem0, gsem1)
    ssem = (ssem0, ssem1)

    pltpu.sync_copy(zeros.at[pl.ds(0, RPT)],
                    acc.at[pl.ds(sid * RPT, RPT)])
    plsc.subcore_barrier()

    def _edges(y_hbm, s_hbm, d_hbm):
        nki = (SUP - 1 - sid) // NT + 1

        def wait_s(rb, iref):
            pltpu.make_async_copy(rows[rb], acc.at[iref], ssem[rb]).wait()

        def wait_g(rb, sref):
            pltpu.make_async_copy(y_hbm.at[sref], rows[rb],
                                  gsem[rb]).wait()

        def body(ko, _):
            for kb in range(2):
                k = ko * 2 + kb
                isx = idx_s[kb]
                idxx = idx_d[kb]
                pidx = idx_d[1 - kb]

                @pl.when(k < nki)
                def _():
                    srow = (k * NT + sid) * 8
                    pltpu.sync_copy(s_hbm.at[pl.ds(srow, 8)], isx)
                    pltpu.sync_copy(d_hbm.at[pl.ds(srow, 8)], idxx)
                    for u in range(8):
                        rb = u % 2
                        orb = 1 - rb
                        if u >= 2:
                            wait_s(rb, idxx.at[u])
                        else:
                            @pl.when(k >= 1)
                            def _():
                                wait_s(rb, idxx.at[u])
                        pltpu.async_copy(y_hbm.at[isx.at[u]], rows[rb],
                                         gsem[rb])
                        if u >= 1:
                            wait_g(orb, isx.at[u - 1])
                            pltpu.async_copy(rows[orb], acc.at[idxx.at[u - 1]],
                                             ssem[orb], add=True)
                        else:
                            @pl.when(k >= 1)
                            def _():
                                wait_g(orb, isx.at[u])
                                pltpu.async_copy(rows[orb],
                                                 acc.at[pidx.at[7]],
                                                 ssem[orb], add=True)
            return 0
        lax.fori_loop(0, SUP_ITERS, body, 0)

        wait_g(1, idx_s[0].at[7])
        for kb in range(2):
            @pl.when((nki - 1) % 2 == kb)
            def _():
                pltpu.async_copy(rows[1], acc.at[idx_d[kb].at[7]],
                                 ssem[1], add=True)
        wait_s(0, idx_d[0].at[6])
        wait_s(1, idx_d[0].at[7])

    @pl.when(cid == 0)
    def _():
        _edges(ya, saa, daa)

    @pl.when(cid == 1)
    def _():
        _edges(yv, svv, dvv)

    plsc.subcore_barrier()
    sl = pl.ds(sid * RPT, RPT)

    @pl.when(cid == 0)
    def _():
        pltpu.sync_copy(acc.at[sl], out_a.at[sl])

    @pl.when(cid == 1)
    def _():
        pltpu.sync_copy(acc.at[sl], out_v.at[sl])


_sc_scatter = pl.kernel(
    _sc_scatter_body,
    out_type=(jax.ShapeDtypeStruct((NPAD, H), jnp.float32),
              jax.ShapeDtypeStruct((NPAD, H), jnp.float32)),
    mesh=plsc.VectorSubcoreMesh(core_axis_name="c", subcore_axis_name="s"),
    scratch_types=[
        pltpu.VMEM_SHARED((NPAD, H), jnp.float32),
        pltpu.VMEM((8, CHUNK), jnp.int32),
        pltpu.VMEM((8, CHUNK), jnp.int32),
        pltpu.VMEM((8, CHUNK), jnp.int32),
        pltpu.VMEM((8, CHUNK), jnp.int32),
        pltpu.VMEM((CHUNK, H), jnp.float32),
        pltpu.VMEM((CHUNK, H), jnp.float32),
        pltpu.SemaphoreType.DMA,
        pltpu.SemaphoreType.DMA,
        pltpu.SemaphoreType.DMA,
        pltpu.SemaphoreType.DMA,
    ],
    compiler_params=pltpu.CompilerParams(needs_layout_passes=False),
    name="sc_edge_scatter",
)


def _sc_gat_body(hs_hbm, ss_hbm, sd_hbm, sva_hbm, dva_hbm, shift_hbm, zeros,
                 num_hbm, den_hbm,
                 acc, den_sh, ss_loc, sd_loc, shiftv, idx_s, idx_d, alpha,
                 rows, zv, sem):
    cid = lax.axis_index("c")
    sid = lax.axis_index("s")
    rpt = NPAD // NT

    pltpu.sync_copy(zeros.at[pl.ds(0, rpt)],
                    acc.at[pl.ds(sid * rpt, rpt)])

    def zbody(i, _):
        zv[pl.ds(i * 16, 16)] = jnp.zeros((16,), jnp.float32)
        return 0
    lax.fori_loop(0, rpt // 16, zbody, 0)
    pltpu.sync_copy(zv, den_sh.at[pl.ds(sid * rpt, rpt)])
    plsc.subcore_barrier()

    @pl.when(cid == 0)
    def _():
        pltpu.sync_copy(ss_hbm, ss_loc)
        pltpu.sync_copy(sd_hbm, sd_loc)
        pltpu.sync_copy(shift_hbm, shiftv)

        def body(i, _):
            chunk = i * NT + sid

            @pl.when(chunk < VA_CHUNKS)
            def _():
                base = chunk * CHUNK
                pltpu.sync_copy(sva_hbm.at[pl.ds(base, CHUNK)], idx_s)
                pltpu.sync_copy(dva_hbm.at[pl.ds(base, CHUNK)], idx_d)
                cp = pltpu.async_copy(hs_hbm.at[idx_s], rows, sem)
                sh = shiftv[...]
                for j in range(CHUNK // 16):
                    ii = idx_s[pl.ds(j * 16, 16)]
                    dd = idx_d[pl.ds(j * 16, 16)]
                    lg = plsc.load_gather(ss_loc, [ii]) \
                        + plsc.load_gather(sd_loc, [dd])
                    lg = jnp.maximum(lg, 0.2 * lg)
                    alpha[pl.ds(j * 16, 16)] = jnp.exp(lg - sh)
                pltpu.sync_copy(alpha, den_sh.at[idx_d], add=True)
                cp.wait()

                def rowgrp(j16, _):
                    a16 = alpha[pl.ds(j16 * 16, 16)]
                    lane = lax.iota(jnp.int32, 16)

                    def one(r2, _):
                        av = jnp.sum(jnp.where(lane == r2, a16, 0.0))
                        avv = jnp.full((16,), av, jnp.float32)
                        r = j16 * 16 + r2
                        for j2 in range(CHUNK // 16):
                            rows[r, pl.ds(j2 * 16, 16)] = (
                                rows[r, pl.ds(j2 * 16, 16)] * avv)
                        return 0
                    lax.fori_loop(0, 16, one, 0)
                    return 0
                lax.fori_loop(0, CHUNK // 16, rowgrp, 0)
                pltpu.sync_copy(rows, acc.at[idx_d], add=True)
            return 0
        lax.fori_loop(0, VA_ITERS, body, 0)

    plsc.subcore_barrier()

    @pl.when(cid == 0)
    def _():
        sl = pl.ds(sid * RPT, RPT)
        pltpu.sync_copy(acc.at[sl], num_hbm.at[sl])
        pltpu.sync_copy(den_sh.at[sl], den_hbm.at[sl])


_sc_gat = pl.kernel(
    _sc_gat_body,
    out_type=(jax.ShapeDtypeStruct((NPAD, H), jnp.float32),
              jax.ShapeDtypeStruct((NPAD,), jnp.float32)),
    mesh=plsc.VectorSubcoreMesh(core_axis_name="c", subcore_axis_name="s"),
    scratch_types=[
        pltpu.VMEM_SHARED((NPAD, H), jnp.float32),
        pltpu.VMEM_SHARED((NPAD,), jnp.float32),
        pltpu.VMEM((NPAD,), jnp.float32),
        pltpu.VMEM((NPAD,), jnp.float32),
        pltpu.VMEM((16,), jnp.float32),
        pltpu.VMEM((CHUNK,), jnp.int32),
        pltpu.VMEM((CHUNK,), jnp.int32),
        pltpu.VMEM((CHUNK,), jnp.float32),
        pltpu.VMEM((CHUNK, H), jnp.float32),
        pltpu.VMEM((NPAD // NT,), jnp.float32),
        pltpu.SemaphoreType.DMA,
    ],
    compiler_params=pltpu.CompilerParams(needs_layout_passes=False),
    name="sc_gat",
)


def _mid_body(ha_ref, hv_ref, baa0, bvv0, lnag, lnab, lnvg, lnvb,
              Waa1, Wvv1, Wgs, Wgd, asv, adv,
              ya1, yv1, hso, ssrc, sdst, mxs, mxd):
    i = pl.program_id(0)
    ha = jnp.maximum(_ln(ha_ref[...] + baa0[...], lnag[...], lnab[...]), 0.0)
    hv = jnp.maximum(_ln(hv_ref[...] + bvv0[...], lnvg[...], lnvb[...]), 0.0)
    ya1[...] = jnp.dot(ha, Waa1[...], preferred_element_type=jnp.float32)
    yv1[...] = jnp.dot(hv, Wvv1[...], preferred_element_type=jnp.float32)
    hs = jnp.dot(hv, Wgs[...], preferred_element_type=jnp.float32)
    hso[...] = hs
    hd = jnp.dot(ha, Wgd[...], preferred_element_type=jnp.float32)
    ss = jnp.dot(hs, asv[...], preferred_element_type=jnp.float32)
    sd = jnp.dot(hd, adv[...], preferred_element_type=jnp.float32)
    ssrc[...] = ss
    sdst[...] = sd

    @pl.when(i == 0)
    def _():
        mxs[...] = jnp.full_like(mxs[...], -1e30)
        mxd[...] = jnp.full_like(mxd[...], -1e30)
    mxs[...] = jnp.maximum(mxs[...], jnp.max(ss))
    mxd[...] = jnp.maximum(mxd[...], jnp.max(sd))


def _mid(ha_raw, hv_raw, baa0, bvv0, lnag, lnab, lnvg, lnvb,
         Waa1, Wvv1, Wgs, Wgd, asv, adv):
    full = lambda shape: pl.BlockSpec(shape, lambda i: (0, 0))
    blk = lambda shape: pl.BlockSpec(shape, lambda i: (i, 0))
    return pl.pallas_call(
        _mid_body,
        grid=(N // BM,),
        in_specs=[blk((BM, H)), blk((BM, H)),
                  full((1, H)), full((1, H)), full((1, H)), full((1, H)),
                  full((1, H)), full((1, H)),
                  full((H, H)), full((H, H)), full((H, H)), full((H, H)),
                  full((H, 1)), full((H, 1))],
        out_specs=[blk((BM, H)), blk((BM, H)), blk((BM, H)),
                   blk((BM, 1)), blk((BM, 1)),
                   full((8, 128)), full((8, 128))],
        out_shape=[jax.ShapeDtypeStruct((N, H), jnp.float32),
                   jax.ShapeDtypeStruct((N, H), jnp.float32),
                   jax.ShapeDtypeStruct((N, H), jnp.float32),
                   jax.ShapeDtypeStruct((N, 1), jnp.float32),
                   jax.ShapeDtypeStruct((N, 1), jnp.float32),
                   jax.ShapeDtypeStruct((8, 128), jnp.float32),
                   jax.ShapeDtypeStruct((8, 128), jnp.float32)],
    )(ha_raw, hv_raw, baa0, bvv0, lnag, lnab, lnvg, lnvb,
      Waa1, Wvv1, Wgs, Wgd, asv, adv)


def _readout(h, wg, batch):
    s = jnp.dot(h, wg, preferred_element_type=jnp.float32)
    gid = lax.broadcasted_iota(jnp.int32, (1, G), 1)
    mask = (batch == gid).astype(jnp.float32)
    M = jnp.max(jnp.where(mask > 0, s, -1e30), axis=0, keepdims=True)
    msel = jnp.sum(mask * M, axis=1, keepdims=True)
    e = jnp.exp(s - msel)
    S = jnp.sum(mask * e, axis=0, keepdims=True)
    ssel = jnp.sum(mask * S, axis=1, keepdims=True)
    gate = e / (ssel + 1e-16)
    w = mask * gate
    return lax.dot_general(w, h, (((0,), (0,)), ((), ())),
                           preferred_element_type=jnp.float32)


def _final_body(h1a, num, den, h1v, baa1, bg, bvv1, ln1ag, ln1ab,
                ln1vg, ln1vb, wga, wgv, ba, bv, Wl, bl, out):
    gat = num[...] / (den[...] + 1e-16)
    ha1 = jnp.maximum(
        _ln(h1a[...] + baa1[...] + gat + bg[...], ln1ag[...], ln1ab[...]),
        0.0)
    hv1 = jnp.maximum(_ln(h1v[...] + bvv1[...], ln1vg[...], ln1vb[...]), 0.0)
    ra = _readout(ha1, wga[...], ba[...])
    rv = _readout(hv1, wgv[...], bv[...])
    Wlv = Wl[...]
    out[...] = (jnp.dot(ra, Wlv[:H, :], preferred_element_type=jnp.float32)
                + jnp.dot(rv, Wlv[H:, :], preferred_element_type=jnp.float32)
                + bl[...])


def _final(h1a, num, den, h1v, baa1, bg, bvv1, ln1ag, ln1ab, ln1vg, ln1vb,
           wga, wgv, ba, bv, Wl, bl):
    return pl.pallas_call(
        _final_body,
        out_shape=jax.ShapeDtypeStruct((G, H), jnp.float32),
    )(h1a, num, den, h1v, baa1, bg, bvv1, ln1ag, ln1ab, ln1vg, ln1vb,
      wga, wgv, ba, bv, Wl, bl)


def kernel(x_audio, x_video, edge_index_aa, edge_index_vv, edge_index_va,
           batch_audio, batch_video, W_aa0, b_aa0, W_vv0, b_vv0, W_aa1,
           b_aa1, W_vv1, b_vv1, W_gs, W_gd, a_s, a_d, b_g, ln_a0_g, ln_a0_b,
           ln_v0_g, ln_v0_b, ln_a1_g, ln_a1_b, ln_v1_g, ln_v1_b, w_gate_a,
           w_gate_v, W_lin, b_lin):
    f32 = jnp.float32
    row = lambda v: v.reshape(1, H)
    col = lambda v: v.reshape(H, 1)
    sva, dva = edge_index_va[0], edge_index_va[1]
    zeros = jnp.zeros((NPAD, H), f32)

    def pad_edges(ei):
        s = jnp.concatenate([ei[0], jnp.zeros((E_P - E,), jnp.int32)])
        d = jnp.concatenate(
            [ei[1], N + (jnp.arange(E_P - E, dtype=jnp.int32) % (NPAD - N))])
        return s.reshape(CROWS, CHUNK), d.reshape(CROWS, CHUNK)

    saa, daa = pad_edges(edge_index_aa)
    svv, dvv = pad_edges(edge_index_vv)

    ya0 = _matmul(x_audio, W_aa0)
    yv0 = _matmul(x_video, W_vv0)
    ha_raw, hv_raw = _sc_scatter(ya0, yv0, saa, daa, svv, dvv, zeros)
    ha_raw, hv_raw = ha_raw[:N], hv_raw[:N]

    (ya1, yv1, hs, ssrc, sdst, mxs, mxd) = _mid(
        ha_raw, hv_raw, row(b_aa0), row(b_vv0), row(ln_a0_g), row(ln_a0_b),
        row(ln_v0_g), row(ln_v0_b), W_aa1, W_vv1, W_gs, W_gd,
        col(a_s), col(a_d))

    h1a_raw, h1v_raw = _sc_scatter(ya1, yv1, saa, daa, svv, dvv, zeros)
    h1a_raw, h1v_raw = h1a_raw[:N], h1v_raw[:N]

    ss_pad = jnp.concatenate([ssrc[:, 0], jnp.zeros((NPAD - N,), f32)])
    sd_pad = jnp.concatenate([sdst[:, 0], jnp.zeros((NPAD - N,), f32)])
    sva_p = jnp.concatenate(
        [sva, jnp.zeros((EVA_PAD - E_VA,), jnp.int32)])
    dva_p = jnp.concatenate(
        [dva, N + (jnp.arange(EVA_PAD - E_VA, dtype=jnp.int32)
                   % (NPAD - N))])
    bound = mxs[0, 0] + mxd[0, 0]
    shift = jnp.full((16,), jnp.maximum(bound, 0.2 * bound), f32)

    num, den = _sc_gat(hs, ss_pad, sd_pad, sva_p, dva_p, shift, zeros)
    num, den = num[:N], den[:N]

    out = _final(
        h1a_raw, num, den.reshape(N, 1), h1v_raw, row(b_aa1), row(b_g),
        row(b_vv1), row(ln_a1_g), row(ln_a1_b), row(ln_v1_g), row(ln_v1_b),
        col(w_gate_a), col(w_gate_v),
        batch_audio.reshape(N, 1), batch_video.reshape(N, 1), W_lin,
        row(b_lin))
    return out


# speedup vs baseline: 8.2776x; 1.0934x over previous
import functools

import jax
import jax.numpy as jnp
from jax import lax
from jax.experimental import pallas as pl
from jax.experimental.pallas import tpu as pltpu
from jax.experimental.pallas import tpu_sc as plsc

N = 10000
H = 128
G = 16
E = 320000
E_VA = 30000
NPAD = 10240
EVA_PAD = 30080
CHUNK = 128
NT = 16
RPT = NPAD // NT
E_P = 320512
CROWS = E_P // CHUNK
SUP = CROWS // 8
SUP_ITERS = (-(-SUP // NT) + 1) // 2
VA_CHUNKS = EVA_PAD // CHUNK
VA_ITERS = -(-VA_CHUNKS // NT)
BM = 1000


def _ln(x, g, b):
    mu = jnp.mean(x, axis=-1, keepdims=True)
    var = jnp.mean((x - mu) ** 2, axis=-1, keepdims=True)
    return (x - mu) / jnp.sqrt(var + 1e-5) * g + b


def _mm_body(xa_ref, wa_ref, xv_ref, wv_ref, oa_ref, ov_ref):
    oa_ref[...] = jnp.dot(xa_ref[...], wa_ref[...],
                          preferred_element_type=jnp.float32)
    ov_ref[...] = jnp.dot(xv_ref[...], wv_ref[...],
                          preferred_element_type=jnp.float32)


def _matmul2(xa, wa, xv, wv):
    blk = pl.BlockSpec((BM, H), lambda i: (i, 0))
    wblk = pl.BlockSpec((H, H), lambda i: (0, 0))
    osd = jax.ShapeDtypeStruct((N, H), jnp.float32)
    return pl.pallas_call(
        _mm_body,
        grid=(N // BM,),
        in_specs=[blk, wblk, blk, wblk],
        out_specs=[blk, blk],
        out_shape=[osd, osd],
    )(xa, wa, xv, wv)


def _sc_scatter_body(ya, yv, saa, daa, svv, dvv, zeros, out_a, out_v,
                     acc, idx_s0, idx_s1, idx_d0, idx_d1, rows0, rows1,
                     gsem0, gsem1, ssem0, ssem1):
    cid = lax.axis_index("c")
    sid = lax.axis_index("s")
    idx_s = (idx_s0, idx_s1)
    idx_d = (idx_d0, idx_d1)
    rows = (rows0, rows1)
    gsem = (gsem0, gsem1)
    ssem = (ssem0, ssem1)

    pltpu.sync_copy(zeros.at[pl.ds(0, RPT)],
                    acc.at[pl.ds(sid * RPT, RPT)])
    plsc.subcore_barrier()

    def _edges(y_hbm, s_hbm, d_hbm):
        nki = (SUP - 1 - sid) // NT + 1

        def wait_s(rb, iref):
            pltpu.make_async_copy(rows[rb], acc.at[iref], ssem[rb]).wait()

        def wait_g(rb, sref):
            pltpu.make_async_copy(y_hbm.at[sref], rows[rb],
                                  gsem[rb]).wait()

        def body(ko, _):
            for kb in range(2):
                k = ko * 2 + kb
                isx = idx_s[kb]
                idxx = idx_d[kb]
                pidx = idx_d[1 - kb]

                @pl.when(k < nki)
                def _():
                    srow = (k * NT + sid) * 8
                    pltpu.sync_copy(s_hbm.at[pl.ds(srow, 8)], isx)
                    pltpu.sync_copy(d_hbm.at[pl.ds(srow, 8)], idxx)
                    for u in range(8):
                        rb = u % 2
                        orb = 1 - rb
                        if u >= 2:
                            wait_s(rb, idxx.at[u])
                        else:
                            @pl.when(k >= 1)
                            def _():
                                wait_s(rb, idxx.at[u])
                        pltpu.async_copy(y_hbm.at[isx.at[u]], rows[rb],
                                         gsem[rb])
                        if u >= 1:
                            wait_g(orb, isx.at[u - 1])
                            pltpu.async_copy(rows[orb], acc.at[idxx.at[u - 1]],
                                             ssem[orb], add=True)
                        else:
                            @pl.when(k >= 1)
                            def _():
                                wait_g(orb, isx.at[u])
                                pltpu.async_copy(rows[orb],
                                                 acc.at[pidx.at[7]],
                                                 ssem[orb], add=True)
            return 0
        lax.fori_loop(0, SUP_ITERS, body, 0)

        wait_g(1, idx_s[0].at[7])
        for kb in range(2):
            @pl.when((nki - 1) % 2 == kb)
            def _():
                pltpu.async_copy(rows[1], acc.at[idx_d[kb].at[7]],
                                 ssem[1], add=True)
        wait_s(0, idx_d[0].at[6])
        wait_s(1, idx_d[0].at[7])

    @pl.when(cid == 0)
    def _():
        _edges(ya, saa, daa)

    @pl.when(cid == 1)
    def _():
        _edges(yv, svv, dvv)

    plsc.subcore_barrier()
    sl = pl.ds(sid * RPT, RPT)

    @pl.when(cid == 0)
    def _():
        pltpu.sync_copy(acc.at[sl], out_a.at[sl])

    @pl.when(cid == 1)
    def _():
        pltpu.sync_copy(acc.at[sl], out_v.at[sl])


_sc_scatter = pl.kernel(
    _sc_scatter_body,
    out_type=(jax.ShapeDtypeStruct((NPAD, H), jnp.float32),
              jax.ShapeDtypeStruct((NPAD, H), jnp.float32)),
    mesh=plsc.VectorSubcoreMesh(core_axis_name="c", subcore_axis_name="s"),
    scratch_types=[
        pltpu.VMEM_SHARED((NPAD, H), jnp.float32),
        pltpu.VMEM((8, CHUNK), jnp.int32),
        pltpu.VMEM((8, CHUNK), jnp.int32),
        pltpu.VMEM((8, CHUNK), jnp.int32),
        pltpu.VMEM((8, CHUNK), jnp.int32),
        pltpu.VMEM((CHUNK, H), jnp.float32),
        pltpu.VMEM((CHUNK, H), jnp.float32),
        pltpu.SemaphoreType.DMA,
        pltpu.SemaphoreType.DMA,
        pltpu.SemaphoreType.DMA,
        pltpu.SemaphoreType.DMA,
    ],
    compiler_params=pltpu.CompilerParams(needs_layout_passes=False),
    name="sc_edge_scatter",
)


def _sc_gat_body(hs_hbm, ss_hbm, sd_hbm, sva_hbm, dva_hbm, shift_hbm, zeros,
                 numa_hbm, numv_hbm, dena_hbm, denv_hbm,
                 acc, den_sh, ss_sh, sd_sh, shiftv,
                 idx_s0, idx_s1, idx_d0, idx_d1, al0, al1, es0, es1,
                 ed0, ed1, rows0, rows1, zv, gsem0, gsem1):
    cid = lax.axis_index("c")
    sid = lax.axis_index("s")
    wid = sid * 2 + cid
    idx_s = (idx_s0, idx_s1)
    idx_d = (idx_d0, idx_d1)
    alpha = (al0, al1)
    esrc = (es0, es1)
    edst = (ed0, ed1)
    rows = (rows0, rows1)
    gsem = (gsem0, gsem1)
    NW = 2 * NT

    pltpu.sync_copy(zeros.at[pl.ds(0, RPT)],
                    acc.at[pl.ds(sid * RPT, RPT)])

    def zbody(i, _):
        zv[pl.ds(i * 16, 16)] = jnp.zeros((16,), jnp.float32)
        return 0
    lax.fori_loop(0, RPT // 16, zbody, 0)
    pltpu.sync_copy(zv, den_sh.at[pl.ds(sid * RPT, RPT)])
    pltpu.sync_copy(shift_hbm, shiftv)

    @pl.when(sid == 0)
    def _():
        pltpu.sync_copy(ss_hbm, ss_sh)
        pltpu.sync_copy(sd_hbm, sd_sh)
    plsc.subcore_barrier()

    nj = (VA_CHUNKS - 1 - wid) // NW + 1

    def ld(j, b):
        base = (j * NW + wid) * CHUNK
        pltpu.sync_copy(sva_hbm.at[pl.ds(base, CHUNK)], idx_s[b])
        pltpu.sync_copy(dva_hbm.at[pl.ds(base, CHUNK)], idx_d[b])
        pltpu.async_copy(hs_hbm.at[idx_s[b]], rows[b], gsem[b])

    ld(0, 0)

    def body(jo, _):
        for b in range(2):
            j = jo * 2 + b

            @pl.when(j < nj)
            def _():
                @pl.when(j + 1 < nj)
                def _():
                    ld(j + 1, 1 - b)
                pltpu.sync_copy(ss_sh.at[idx_s[b]], esrc[b])
                pltpu.sync_copy(sd_sh.at[idx_d[b]], edst[b])
                sh = shiftv[...]
                for u in range(CHUNK // 16):
                    lg = esrc[b][pl.ds(u * 16, 16)] \
                        + edst[b][pl.ds(u * 16, 16)]
                    lg = jnp.maximum(lg, 0.2 * lg)
                    alpha[b][pl.ds(u * 16, 16)] = jnp.exp(lg - sh)
                pltpu.sync_copy(alpha[b], den_sh.at[idx_d[b]], add=True)
                pltpu.make_async_copy(hs_hbm.at[idx_s[b]], rows[b],
                                      gsem[b]).wait()

                def rowgrp(u16, _):
                    a16 = alpha[b][pl.ds(u16 * 16, 16)]

                    def one(r2, _):
                        avv = jnp.take_along_axis(
                            a16, jnp.full((16,), r2, jnp.int32), axis=0)
                        r = u16 * 16 + r2
                        for u2 in range(CHUNK // 16):
                            rows[b][r, pl.ds(u2 * 16, 16)] = (
                                rows[b][r, pl.ds(u2 * 16, 16)] * avv)
                        return 0
                    lax.fori_loop(0, 16, one, 0)
                    return 0
                lax.fori_loop(0, CHUNK // 16, rowgrp, 0)
                pltpu.sync_copy(rows[b], acc.at[idx_d[b]], add=True)
        return 0
    lax.fori_loop(0, (VA_ITERS + 1) // 2, body, 0)

    plsc.subcore_barrier()
    sl = pl.ds(sid * RPT, RPT)

    @pl.when(cid == 0)
    def _():
        pltpu.sync_copy(acc.at[sl], numa_hbm.at[sl])
        pltpu.sync_copy(den_sh.at[sl], dena_hbm.at[sl])

    @pl.when(cid == 1)
    def _():
        pltpu.sync_copy(acc.at[sl], numv_hbm.at[sl])
        pltpu.sync_copy(den_sh.at[sl], denv_hbm.at[sl])


_sc_gat = pl.kernel(
    _sc_gat_body,
    out_type=(jax.ShapeDtypeStruct((NPAD, H), jnp.float32),
              jax.ShapeDtypeStruct((NPAD, H), jnp.float32),
              jax.ShapeDtypeStruct((NPAD,), jnp.float32),
              jax.ShapeDtypeStruct((NPAD,), jnp.float32)),
    mesh=plsc.VectorSubcoreMesh(core_axis_name="c", subcore_axis_name="s"),
    scratch_types=[
        pltpu.VMEM_SHARED((NPAD, H), jnp.float32),
        pltpu.VMEM_SHARED((NPAD,), jnp.float32),
        pltpu.VMEM_SHARED((NPAD,), jnp.float32),
        pltpu.VMEM_SHARED((NPAD,), jnp.float32),
        pltpu.VMEM((16,), jnp.float32),
        pltpu.VMEM((CHUNK,), jnp.int32),
        pltpu.VMEM((CHUNK,), jnp.int32),
        pltpu.VMEM((CHUNK,), jnp.int32),
        pltpu.VMEM((CHUNK,), jnp.int32),
        pltpu.VMEM((CHUNK,), jnp.float32),
        pltpu.VMEM((CHUNK,), jnp.float32),
        pltpu.VMEM((CHUNK,), jnp.float32),
        pltpu.VMEM((CHUNK,), jnp.float32),
        pltpu.VMEM((CHUNK,), jnp.float32),
        pltpu.VMEM((CHUNK,), jnp.float32),
        pltpu.VMEM((CHUNK, H), jnp.float32),
        pltpu.VMEM((CHUNK, H), jnp.float32),
        pltpu.VMEM((RPT,), jnp.float32),
        pltpu.SemaphoreType.DMA,
        pltpu.SemaphoreType.DMA,
    ],
    compiler_params=pltpu.CompilerParams(needs_layout_passes=False),
    name="sc_gat",
)


def _mid_body(ha_ref, hv_ref, baa0, bvv0, lnag, lnab, lnvg, lnvb,
              Waa1, Wvv1, Wgs, Wgd, asv, adv,
              ya1, yv1, hso, ssrc, sdst, mxs, mxd):
    i = pl.program_id(0)
    ha = jnp.maximum(_ln(ha_ref[...] + baa0[...], lnag[...], lnab[...]), 0.0)
    hv = jnp.maximum(_ln(hv_ref[...] + bvv0[...], lnvg[...], lnvb[...]), 0.0)
    ya1[...] = jnp.dot(ha, Waa1[...], preferred_element_type=jnp.float32)
    yv1[...] = jnp.dot(hv, Wvv1[...], preferred_element_type=jnp.float32)
    hs = jnp.dot(hv, Wgs[...], preferred_element_type=jnp.float32)
    hso[...] = hs
    hd = jnp.dot(ha, Wgd[...], preferred_element_type=jnp.float32)
    ss = jnp.dot(hs, asv[...], preferred_element_type=jnp.float32)
    sd = jnp.dot(hd, adv[...], preferred_element_type=jnp.float32)
    ssrc[...] = ss
    sdst[...] = sd

    @pl.when(i == 0)
    def _():
        mxs[...] = jnp.full_like(mxs[...], -1e30)
        mxd[...] = jnp.full_like(mxd[...], -1e30)
    mxs[...] = jnp.maximum(mxs[...], jnp.max(ss))
    mxd[...] = jnp.maximum(mxd[...], jnp.max(sd))


def _mid(ha_raw, hv_raw, baa0, bvv0, lnag, lnab, lnvg, lnvb,
         Waa1, Wvv1, Wgs, Wgd, asv, adv):
    full = lambda shape: pl.BlockSpec(shape, lambda i: (0, 0))
    blk = lambda shape: pl.BlockSpec(shape, lambda i: (i, 0))
    return pl.pallas_call(
        _mid_body,
        grid=(N // BM,),
        in_specs=[blk((BM, H)), blk((BM, H)),
                  full((1, H)), full((1, H)), full((1, H)), full((1, H)),
                  full((1, H)), full((1, H)),
                  full((H, H)), full((H, H)), full((H, H)), full((H, H)),
                  full((H, 1)), full((H, 1))],
        out_specs=[blk((BM, H)), blk((BM, H)), blk((BM, H)),
                   blk((BM, 1)), blk((BM, 1)),
                   full((8, 128)), full((8, 128))],
        out_shape=[jax.ShapeDtypeStruct((N, H), jnp.float32),
                   jax.ShapeDtypeStruct((N, H), jnp.float32),
                   jax.ShapeDtypeStruct((N, H), jnp.float32),
                   jax.ShapeDtypeStruct((N, 1), jnp.float32),
                   jax.ShapeDtypeStruct((N, 1), jnp.float32),
                   jax.ShapeDtypeStruct((8, 128), jnp.float32),
                   jax.ShapeDtypeStruct((8, 128), jnp.float32)],
    )(ha_raw, hv_raw, baa0, bvv0, lnag, lnab, lnvg, lnvb,
      Waa1, Wvv1, Wgs, Wgd, asv, adv)


def _mask_of(batch):
    gid = lax.broadcasted_iota(jnp.int32, (1, G), 1)
    return (batch == gid).astype(jnp.float32)


def _final1_body(h1a, numa, numv, dena, denv, h1v, baa1, bg, bvv1, ln1ag,
                 ln1ab, ln1vg, ln1vb, wga, wgv, ba, bv,
                 ha1o, hv1o, sao, svo, Ma, Mv):
    i = pl.program_id(0)
    gat = (numa[...] + numv[...]) / (dena[...] + denv[...] + 1e-16)
    ha1 = jnp.maximum(
        _ln(h1a[...] + baa1[...] + gat + bg[...], ln1ag[...], ln1ab[...]),
        0.0)
    hv1 = jnp.maximum(_ln(h1v[...] + bvv1[...], ln1vg[...], ln1vb[...]), 0.0)
    ha1o[...] = ha1
    hv1o[...] = hv1
    sa = jnp.dot(ha1, wga[...], preferred_element_type=jnp.float32)
    sv = jnp.dot(hv1, wgv[...], preferred_element_type=jnp.float32)
    sao[...] = sa
    svo[...] = sv

    @pl.when(i == 0)
    def _():
        Ma[...] = jnp.full_like(Ma[...], -1e30)
        Mv[...] = jnp.full_like(Mv[...], -1e30)
    bma = jnp.max(jnp.where(_mask_of(ba[...]) > 0, sa, -1e30), axis=0,
                  keepdims=True)
    bmv = jnp.max(jnp.where(_mask_of(bv[...]) > 0, sv, -1e30), axis=0,
                  keepdims=True)
    Ma[...] = jnp.maximum(Ma[...], jnp.broadcast_to(bma, (8, G)))
    Mv[...] = jnp.maximum(Mv[...], jnp.broadcast_to(bmv, (8, G)))


def _final2_body(ha1, hv1, sa, sv, ba, bv, Ma, Mv, Wl, bl,
                 Ua, Sa, Uv, Sv, out):
    i = pl.program_id(0)

    @pl.when(i == 0)
    def _():
        Ua[...] = jnp.zeros_like(Ua[...])
        Sa[...] = jnp.zeros_like(Sa[...])
        Uv[...] = jnp.zeros_like(Uv[...])
        Sv[...] = jnp.zeros_like(Sv[...])

    def acc(h, s, batch, M, U, S):
        mask = _mask_of(batch)
        msel = jnp.sum(mask * M[0:1, :], axis=1, keepdims=True)
        e = jnp.exp(s - msel)
        S[...] += jnp.broadcast_to(
            jnp.sum(mask * e, axis=0, keepdims=True), (8, G))
        U[...] += lax.dot_general(mask * e, h, (((0,), (0,)), ((), ())),
                                  preferred_element_type=jnp.float32)
    acc(ha1[...], sa[...], ba[...], Ma[...], Ua, Sa)
    acc(hv1[...], sv[...], bv[...], Mv[...], Uv, Sv)

    @pl.when(i == N // BM - 1)
    def _():
        sca = jnp.transpose(Sa[...][0:1, :]) + 1e-16
        scv = jnp.transpose(Sv[...][0:1, :]) + 1e-16
        ra = Ua[...] / sca
        rv = Uv[...] / scv
        Wlv = Wl[...]
        out[...] = (jnp.dot(ra, Wlv[:H, :],
                            preferred_element_type=jnp.float32)
                    + jnp.dot(rv, Wlv[H:, :],
                              preferred_element_type=jnp.float32)
                    + bl[...])


def _final(h1a, numa, numv, dena, denv, h1v, baa1, bg, bvv1, ln1ag, ln1ab,
           ln1vg, ln1vb, wga, wgv, ba, bv, Wl, bl):
    blk = lambda shape: pl.BlockSpec(shape, lambda i: (i, 0))
    full = lambda shape: pl.BlockSpec(shape, lambda i: (0, 0))
    nh = blk((BM, H))
    n1 = blk((BM, 1))
    rh = full((1, H))
    ch = full((H, 1))
    m8 = full((8, G))
    sd = jax.ShapeDtypeStruct
    ha1, hv1, sa, sv, Ma, Mv = pl.pallas_call(
        _final1_body,
        grid=(N // BM,),
        in_specs=[nh, nh, nh, n1, n1, nh, rh, rh, rh, rh, rh,
                  rh, rh, ch, ch, n1, n1],
        out_specs=[nh, nh, n1, n1, m8, m8],
        out_shape=[sd((N, H), jnp.float32), sd((N, H), jnp.float32),
                   sd((N, 1), jnp.float32), sd((N, 1), jnp.float32),
                   sd((8, G), jnp.float32), sd((8, G), jnp.float32)],
    )(h1a, numa, numv, dena, denv, h1v, baa1, bg, bvv1, ln1ag, ln1ab,
      ln1vg, ln1vb, wga, wgv, ba, bv)
    _, _, _, _, out = pl.pallas_call(
        _final2_body,
        grid=(N // BM,),
        in_specs=[nh, nh, n1, n1, n1, n1, m8, m8,
                  full((2 * H, H)), rh],
        out_specs=[full((G, H)), m8, full((G, H)), m8, full((G, H))],
        out_shape=[sd((G, H), jnp.float32), sd((8, G), jnp.float32),
                   sd((G, H), jnp.float32), sd((8, G), jnp.float32),
                   sd((G, H), jnp.float32)],
    )(ha1, hv1, sa, sv, ba, bv, Ma, Mv, Wl, bl)
    return out


def kernel(x_audio, x_video, edge_index_aa, edge_index_vv, edge_index_va,
           batch_audio, batch_video, W_aa0, b_aa0, W_vv0, b_vv0, W_aa1,
           b_aa1, W_vv1, b_vv1, W_gs, W_gd, a_s, a_d, b_g, ln_a0_g, ln_a0_b,
           ln_v0_g, ln_v0_b, ln_a1_g, ln_a1_b, ln_v1_g, ln_v1_b, w_gate_a,
           w_gate_v, W_lin, b_lin):
    f32 = jnp.float32
    row = lambda v: v.reshape(1, H)
    col = lambda v: v.reshape(H, 1)
    sva, dva = edge_index_va[0], edge_index_va[1]
    zeros = jnp.zeros((RPT, H), f32)

    def pad_edges(ei):
        s = jnp.concatenate([ei[0], jnp.zeros((E_P - E,), jnp.int32)])
        d = jnp.concatenate(
            [ei[1], N + (jnp.arange(E_P - E, dtype=jnp.int32) % (NPAD - N))])
        return s.reshape(CROWS, CHUNK), d.reshape(CROWS, CHUNK)

    saa, daa = pad_edges(edge_index_aa)
    svv, dvv = pad_edges(edge_index_vv)

    ya0, yv0 = _matmul2(x_audio, W_aa0, x_video, W_vv0)
    ha_raw, hv_raw = _sc_scatter(ya0, yv0, saa, daa, svv, dvv, zeros)

    (ya1, yv1, hs, ssrc, sdst, mxs, mxd) = _mid(
        ha_raw, hv_raw, row(b_aa0), row(b_vv0), row(ln_a0_g), row(ln_a0_b),
        row(ln_v0_g), row(ln_v0_b), W_aa1, W_vv1, W_gs, W_gd,
        col(a_s), col(a_d))

    h1a_raw, h1v_raw = _sc_scatter(ya1, yv1, saa, daa, svv, dvv, zeros)

    ss_pad = jnp.concatenate([ssrc[:, 0], jnp.zeros((NPAD - N,), f32)])
    sd_pad = jnp.concatenate([sdst[:, 0], jnp.zeros((NPAD - N,), f32)])
    sva_p = jnp.concatenate(
        [sva, jnp.zeros((EVA_PAD - E_VA,), jnp.int32)])
    dva_p = jnp.concatenate(
        [dva, N + (jnp.arange(EVA_PAD - E_VA, dtype=jnp.int32)
                   % (NPAD - N))])
    bound = mxs[0, 0] + mxd[0, 0]
    shift = jnp.full((16,), jnp.maximum(bound, 0.2 * bound), f32)

    numa, numv, dena, denv = _sc_gat(hs, ss_pad, sd_pad, sva_p, dva_p,
                                     shift, zeros)

    out = _final(
        h1a_raw, numa, numv, dena.reshape(NPAD, 1), denv.reshape(NPAD, 1),
        h1v_raw, row(b_aa1), row(b_g), row(b_vv1), row(ln_a1_g),
        row(ln_a1_b), row(ln_v1_g), row(ln_v1_b),
        col(w_gate_a), col(w_gate_v),
        batch_audio.reshape(N, 1), batch_video.reshape(N, 1), W_lin,
        row(b_lin))
    return out
